# baseline (speedup 1.0000x reference)
import functools

import jax
import jax.numpy as jnp
from jax.experimental import pallas as pl
from jax.experimental.pallas import tpu as pltpu
from jax.experimental.pallas import tpu_sc as plsc

_B = 1024
_THRESH = 0.5

_NC, _NS, _L = 2, 16, 16
_NW = _NC * _NS


def _gather_body(bflat, sflat, orderp, bt, box_v, sco_v, idx_v,
                 x1_v, y1_v, x2_v, y2_v, ar_v, sc_v):
    wid = jax.lax.axis_index("s") * _NC + jax.lax.axis_index("c")
    npad = bt.shape[0] // 8
    rows = npad // _NW
    base = wid * rows
    pltpu.sync_copy(bflat, box_v)
    pltpu.sync_copy(sflat, sco_v)
    pltpu.sync_copy(orderp.at[pl.ds(base, rows)], idx_v)
    for k in range(rows // _L):
        sl = pl.ds(k * _L, _L)
        idx = idx_v[sl]
        i4 = idx * 4
        x1 = plsc.load_gather(box_v, [i4])
        y1 = plsc.load_gather(box_v, [i4 + 1])
        x2 = plsc.load_gather(box_v, [i4 + 2])
        y2 = plsc.load_gather(box_v, [i4 + 3])
        x1_v[sl] = x1
        y1_v[sl] = y1
        x2_v[sl] = x2
        y2_v[sl] = y2
        ar_v[sl] = (x2 - x1) * (y2 - y1)
        sc_v[sl] = plsc.load_gather(sco_v, [idx])
    pltpu.sync_copy(x1_v, bt.at[pl.ds(0 * npad + base, rows)])
    pltpu.sync_copy(y1_v, bt.at[pl.ds(1 * npad + base, rows)])
    pltpu.sync_copy(x2_v, bt.at[pl.ds(2 * npad + base, rows)])
    pltpu.sync_copy(y2_v, bt.at[pl.ds(3 * npad + base, rows)])
    pltpu.sync_copy(ar_v, bt.at[pl.ds(4 * npad + base, rows)])
    pltpu.sync_copy(sc_v, bt.at[pl.ds(5 * npad + base, rows)])


def _sc_sorted_gather(bflat, sflat, orderp, npad):
    rows = npad // _NW
    body = functools.partial(
        pl.kernel,
        out_type=jax.ShapeDtypeStruct((8 * npad,), jnp.float32),
        mesh=plsc.VectorSubcoreMesh(core_axis_name="c", subcore_axis_name="s"),
        compiler_params=pltpu.CompilerParams(needs_layout_passes=False),
        scratch_types=[
            pltpu.VMEM(bflat.shape, jnp.float32),
            pltpu.VMEM(sflat.shape, jnp.float32),
            pltpu.VMEM((rows,), jnp.int32),
        ] + [pltpu.VMEM((rows,), jnp.float32)] * 6,
    )(_gather_body)
    return body(bflat, sflat, orderp)


def _nms_kernel(bt_ref, btc_ref, out_ref, keep_ref, m_ref, cb_ref):
    npad = bt_ref.shape[1]
    nb = npad // _B

    def iou_tile(jb):
        sx1 = bt_ref[0:1, pl.ds(jb, _B)]
        sy1 = bt_ref[1:2, pl.ds(jb, _B)]
        sx2 = bt_ref[2:3, pl.ds(jb, _B)]
        sy2 = bt_ref[3:4, pl.ds(jb, _B)]
        sar = bt_ref[4:5, pl.ds(jb, _B)]
        xx1 = jnp.maximum(cb_ref[0], sx1)
        yy1 = jnp.maximum(cb_ref[1], sy1)
        xx2 = jnp.minimum(cb_ref[2], sx2)
        yy2 = jnp.minimum(cb_ref[3], sy2)
        iw = jnp.maximum(xx2 - xx1, 0.0)
        ih = jnp.maximum(yy2 - yy1, 0.0)
        inter = iw * ih
        union = (cb_ref[4] + sar) - inter
        return inter / (union + 1e-9)

    def outer(i, carry):
        base = i * _B
        for c in range(5):
            cb_ref[c] = jnp.broadcast_to(
                btc_ref[pl.ds(base, _B), c:c + 1], (_B, _B))

        def inner(j, cnt):
            jb = j * _B
            iou = iou_tile(jb)
            over = (iou > _THRESH).astype(jnp.float32)
            kj = keep_ref[0:1, pl.ds(jb, _B)]
            return cnt + jax.lax.dot_general(
                kj, over, (((1,), (1,)), ((), ())),
                preferred_element_type=jnp.float32)

        cnt = jax.lax.fori_loop(0, i, inner, jnp.zeros((1, _B), jnp.float32))
        init = (cnt < 0.5).astype(jnp.float32)

        iou_ii = iou_tile(base)
        riota = jax.lax.broadcasted_iota(jnp.int32, (_B, _B), 0)
        ciota = jax.lax.broadcasted_iota(jnp.int32, (_B, _B), 1)
        m_ref[:, :] = jnp.where((iou_ii > _THRESH) & (ciota < riota), 1.0, 0.0)

        def w_cond(st):
            t, _, changed = st
            return changed & (t < _B)

        def w_body(st):
            t, k, _ = st
            cnt2 = jax.lax.dot_general(
                k, m_ref[:, :], (((1,), (1,)), ((), ())),
                preferred_element_type=jnp.float32)
            k2 = jnp.where(cnt2 < 0.5, init, 0.0)
            return t + 1, k2, jnp.any(k2 != k)

        _, k_fin, _ = jax.lax.while_loop(
            w_cond, w_body, (jnp.int32(0), init, True))
        keep_ref[0:1, pl.ds(base, _B)] = k_fin
        return carry

    jax.lax.fori_loop(0, nb, outer, jnp.int32(0))

    keep = keep_ref[0:1, :]
    s = bt_ref[5:6, :]
    s_adj = jnp.where(s >= 1.0, s - 1.0, s)
    out_ref[0:1, :] = s_adj * keep
    out_ref[1:2, :] = bt_ref[0:1, :] * keep
    out_ref[2:3, :] = bt_ref[1:2, :] * keep
    out_ref[3:4, :] = bt_ref[2:3, :] * keep
    out_ref[4:5, :] = bt_ref[3:4, :] * keep
    out_ref[5:8, :] = jnp.zeros((3, npad), jnp.float32)


def kernel(boxes, scores):
    n = scores.shape[0]
    order = jnp.argsort(-scores).astype(jnp.int32)
    npad = ((n + _B - 1) // _B) * _B
    orderp = jnp.concatenate(
        [order, jnp.full((npad - n,), n, jnp.int32)])
    bflat = jnp.concatenate(
        [boxes.reshape(-1), jnp.zeros((32,), jnp.float32)])
    sflat = jnp.concatenate([scores, jnp.zeros((8,), jnp.float32)])
    bt = _sc_sorted_gather(bflat, sflat, orderp, npad).reshape(8, npad)
    btc = bt.T

    out = pl.pallas_call(
        _nms_kernel,
        out_shape=jax.ShapeDtypeStruct((8, npad), jnp.float32),
        scratch_shapes=[
            pltpu.VMEM((8, npad), jnp.float32),
            pltpu.VMEM((_B, _B), jnp.float32),
            pltpu.VMEM((5, _B, _B), jnp.float32),
        ],
    )(bt, btc)
    return out[:5, :n].T


# speedup vs baseline: 1.0347x; 1.0347x over previous
import functools

import jax
import jax.numpy as jnp
from jax.experimental import pallas as pl
from jax.experimental.pallas import tpu as pltpu
from jax.experimental.pallas import tpu_sc as plsc

_B = 640
_THRESH = 0.5

_NC, _NS, _L = 2, 16, 16
_NW = _NC * _NS


def _gather_body(bflat, sflat, orderp, bt, box_v, sco_v, idx_v,
                 x1_v, y1_v, x2_v, y2_v, ar_v, sc_v):
    wid = jax.lax.axis_index("s") * _NC + jax.lax.axis_index("c")
    npad = bt.shape[0] // 8
    rows = npad // _NW
    base = wid * rows
    pltpu.sync_copy(bflat, box_v)
    pltpu.sync_copy(sflat, sco_v)
    pltpu.sync_copy(orderp.at[pl.ds(base, rows)], idx_v)
    for k in range(rows // _L):
        sl = pl.ds(k * _L, _L)
        idx = idx_v[sl]
        i4 = idx * 4
        x1 = plsc.load_gather(box_v, [i4])
        y1 = plsc.load_gather(box_v, [i4 + 1])
        x2 = plsc.load_gather(box_v, [i4 + 2])
        y2 = plsc.load_gather(box_v, [i4 + 3])
        x1_v[sl] = x1
        y1_v[sl] = y1
        x2_v[sl] = x2
        y2_v[sl] = y2
        ar_v[sl] = (x2 - x1) * (y2 - y1)
        sc_v[sl] = plsc.load_gather(sco_v, [idx])
    pltpu.sync_copy(x1_v, bt.at[pl.ds(0 * npad + base, rows)])
    pltpu.sync_copy(y1_v, bt.at[pl.ds(1 * npad + base, rows)])
    pltpu.sync_copy(x2_v, bt.at[pl.ds(2 * npad + base, rows)])
    pltpu.sync_copy(y2_v, bt.at[pl.ds(3 * npad + base, rows)])
    pltpu.sync_copy(ar_v, bt.at[pl.ds(4 * npad + base, rows)])
    pltpu.sync_copy(sc_v, bt.at[pl.ds(5 * npad + base, rows)])


def _sc_sorted_gather(bflat, sflat, orderp, npad):
    rows = npad // _NW
    body = functools.partial(
        pl.kernel,
        out_type=jax.ShapeDtypeStruct((8 * npad,), jnp.float32),
        mesh=plsc.VectorSubcoreMesh(core_axis_name="c", subcore_axis_name="s"),
        compiler_params=pltpu.CompilerParams(needs_layout_passes=False),
        scratch_types=[
            pltpu.VMEM(bflat.shape, jnp.float32),
            pltpu.VMEM(sflat.shape, jnp.float32),
            pltpu.VMEM((rows,), jnp.int32),
        ] + [pltpu.VMEM((rows,), jnp.float32)] * 6,
    )(_gather_body)
    return body(bflat, sflat, orderp)


def _nms_kernel(bt_ref, btc_ref, out_ref, keep_ref, m_ref, cb_ref):
    npad = bt_ref.shape[1]
    nb = npad // _B

    def iou_tile(jb):
        sx1 = bt_ref[0:1, pl.ds(jb, _B)]
        sy1 = bt_ref[1:2, pl.ds(jb, _B)]
        sx2 = bt_ref[2:3, pl.ds(jb, _B)]
        sy2 = bt_ref[3:4, pl.ds(jb, _B)]
        sar = bt_ref[4:5, pl.ds(jb, _B)]
        xx1 = jnp.maximum(cb_ref[0], sx1)
        yy1 = jnp.maximum(cb_ref[1], sy1)
        xx2 = jnp.minimum(cb_ref[2], sx2)
        yy2 = jnp.minimum(cb_ref[3], sy2)
        iw = jnp.maximum(xx2 - xx1, 0.0)
        ih = jnp.maximum(yy2 - yy1, 0.0)
        inter = iw * ih
        union = (cb_ref[4] + sar) - inter
        return inter / (union + 1e-9)

    def outer(i, carry):
        base = i * _B
        for c in range(5):
            cb_ref[c] = jnp.broadcast_to(
                btc_ref[pl.ds(base, _B), c:c + 1], (_B, _B))

        def inner(j, cnt):
            jb = j * _B
            iou = iou_tile(jb)
            over = (iou > _THRESH).astype(jnp.float32)
            kj = keep_ref[0:1, pl.ds(jb, _B)]
            return cnt + jax.lax.dot_general(
                kj, over, (((1,), (1,)), ((), ())),
                preferred_element_type=jnp.float32)

        cnt = jax.lax.fori_loop(0, i, inner, jnp.zeros((1, _B), jnp.float32))
        init = (cnt < 0.5).astype(jnp.float32)

        iou_ii = iou_tile(base)
        riota = jax.lax.broadcasted_iota(jnp.int32, (_B, _B), 0)
        ciota = jax.lax.broadcasted_iota(jnp.int32, (_B, _B), 1)
        m_ref[:, :] = jnp.where((iou_ii > _THRESH) & (ciota < riota), 1.0, 0.0)

        def w_cond(st):
            t, _, changed = st
            return changed & (t < _B)

        def w_body(st):
            t, k, _ = st
            cnt2 = jax.lax.dot_general(
                k, m_ref[:, :], (((1,), (1,)), ((), ())),
                preferred_element_type=jnp.float32)
            k2 = jnp.where(cnt2 < 0.5, init, 0.0)
            return t + 1, k2, jnp.any(k2 != k)

        _, k_fin, _ = jax.lax.while_loop(
            w_cond, w_body, (jnp.int32(0), init, True))
        keep_ref[0:1, pl.ds(base, _B)] = k_fin
        return carry

    jax.lax.fori_loop(0, nb, outer, jnp.int32(0))

    keep = keep_ref[0:1, :]
    s = bt_ref[5:6, :]
    s_adj = jnp.where(s >= 1.0, s - 1.0, s)
    out_ref[0:1, :] = s_adj * keep
    out_ref[1:2, :] = bt_ref[0:1, :] * keep
    out_ref[2:3, :] = bt_ref[1:2, :] * keep
    out_ref[3:4, :] = bt_ref[2:3, :] * keep
    out_ref[4:5, :] = bt_ref[3:4, :] * keep
    out_ref[5:8, :] = jnp.zeros((3, npad), jnp.float32)


def kernel(boxes, scores):
    n = scores.shape[0]
    order = jnp.argsort(-scores).astype(jnp.int32)
    npad = ((n + _B - 1) // _B) * _B
    orderp = jnp.concatenate(
        [order, jnp.full((npad - n,), n, jnp.int32)])
    bflat = jnp.concatenate(
        [boxes.reshape(-1), jnp.zeros((32,), jnp.float32)])
    sflat = jnp.concatenate([scores, jnp.zeros((8,), jnp.float32)])
    bt = _sc_sorted_gather(bflat, sflat, orderp, npad).reshape(8, npad)
    btc = bt.T

    out = pl.pallas_call(
        _nms_kernel,
        out_shape=jax.ShapeDtypeStruct((8, npad), jnp.float32),
        scratch_shapes=[
            pltpu.VMEM((8, npad), jnp.float32),
            pltpu.VMEM((_B, _B), jnp.float32),
            pltpu.VMEM((5, _B, _B), jnp.float32),
        ],
    )(bt, btc)
    return out[:5, :n].T
